# baseline (speedup 1.0000x reference)
import jax
import jax.numpy as jnp
from jax.experimental import pallas as pl
from jax.experimental.pallas import tpu as pltpu

NUM_NODES = 10000
NODE_DIM = 128
NUM_NETS = 4096

PACK = 8
PACKED_ROWS = NUM_NETS // PACK
R_BLOCK = 256


def _mean_net_kernel(pk_ref, emb_ref, out_ref):
    pki = pk_ref[...].astype(jnp.int32)
    emb = emb_ref[...]
    for k in range(PACK):
        mk = (pki & (1 << k)).astype(jnp.bfloat16)
        sums = jax.lax.dot_general(
            mk, emb,
            dimension_numbers=(((1,), (0,)), ((), ())),
            preferred_element_type=jnp.float32,
        )
        counts = jnp.sum(mk, axis=1, keepdims=True, dtype=jnp.float32)
        out_ref[:, k, :] = jnp.where(counts > 0, sums / jnp.maximum(counts, 1.0), 0.0)


def kernel(node_embeddings, attention_mask):
    emb_bf16 = node_embeddings.astype(jnp.bfloat16)
    mm = attention_mask.reshape(PACKED_ROWS, PACK, NUM_NODES)
    weights = (jnp.uint8(1) << jnp.arange(PACK, dtype=jnp.uint8))[None, :, None]
    pk = jnp.sum(mm.astype(jnp.uint8) * weights, axis=1, dtype=jnp.uint8)
    pk = pk.astype(jnp.int8)

    grid = (PACKED_ROWS // R_BLOCK,)
    out = pl.pallas_call(
        _mean_net_kernel,
        grid=grid,
        in_specs=[
            pl.BlockSpec((R_BLOCK, NUM_NODES), lambda i: (i, 0)),
            pl.BlockSpec((NUM_NODES, NODE_DIM), lambda i: (0, 0)),
        ],
        out_specs=pl.BlockSpec((R_BLOCK, PACK, NODE_DIM), lambda i: (i, 0, 0)),
        out_shape=jax.ShapeDtypeStruct((PACKED_ROWS, PACK, NODE_DIM), jnp.float32),
        compiler_params=pltpu.CompilerParams(
            dimension_semantics=("parallel",),
        ),
    )(pk, emb_bf16)
    return out.reshape(NUM_NETS, NODE_DIM)


# speedup vs baseline: 2.6716x; 2.6716x over previous
import jax
import jax.numpy as jnp
from jax.experimental import pallas as pl
from jax.experimental.pallas import tpu as pltpu

NUM_NODES = 10000
NODE_DIM = 128
NUM_NETS = 4096

M_BLOCK = 256


def _mean_net_kernel(mask_ref, emb_ref, out_ref):
    mb = mask_ref[...].astype(jnp.bfloat16)
    sums = jax.lax.dot_general(
        mb, emb_ref[...],
        dimension_numbers=(((1,), (0,)), ((), ())),
        preferred_element_type=jnp.float32,
    )
    counts = jnp.sum(mb, axis=1, keepdims=True, dtype=jnp.float32)
    out_ref[...] = jnp.where(counts > 0, sums / jnp.maximum(counts, 1.0), 0.0)


def kernel(node_embeddings, attention_mask):
    emb_bf16 = node_embeddings.astype(jnp.bfloat16)
    mask_i8 = attention_mask.astype(jnp.int8)
    grid = (NUM_NETS // M_BLOCK,)
    return pl.pallas_call(
        _mean_net_kernel,
        grid=grid,
        in_specs=[
            pl.BlockSpec((M_BLOCK, NUM_NODES), lambda i: (i, 0)),
            pl.BlockSpec((NUM_NODES, NODE_DIM), lambda i: (0, 0)),
        ],
        out_specs=pl.BlockSpec((M_BLOCK, NODE_DIM), lambda i: (i, 0)),
        out_shape=jax.ShapeDtypeStruct((NUM_NETS, NODE_DIM), jnp.float32),
        compiler_params=pltpu.CompilerParams(
            dimension_semantics=("parallel",),
        ),
    )(mask_i8, emb_bf16)


# speedup vs baseline: 2.9192x; 1.0927x over previous
import jax
import jax.numpy as jnp
from jax.experimental import pallas as pl
from jax.experimental.pallas import tpu as pltpu

NUM_NODES = 10000
NODE_DIM = 128
NUM_NETS = 4096

M_BLOCK = 256


def _mean_net_kernel(mask_ref, emb_ref, out_ref):
    mb = mask_ref[...].astype(jnp.bfloat16)
    sums = jax.lax.dot_general(
        mb, emb_ref[...],
        dimension_numbers=(((1,), (0,)), ((), ())),
        preferred_element_type=jnp.float32,
    )
    counts = jnp.sum(mb, axis=1, keepdims=True, dtype=jnp.float32)
    out_ref[...] = jnp.where(counts > 0, sums / jnp.maximum(counts, 1.0), 0.0)


def kernel(node_embeddings, attention_mask):
    emb_bf16 = node_embeddings.astype(jnp.bfloat16)
    mask_i8 = attention_mask.astype(jnp.int4)
    grid = (NUM_NETS // M_BLOCK,)
    return pl.pallas_call(
        _mean_net_kernel,
        grid=grid,
        in_specs=[
            pl.BlockSpec((M_BLOCK, NUM_NODES), lambda i: (i, 0)),
            pl.BlockSpec((NUM_NODES, NODE_DIM), lambda i: (0, 0)),
        ],
        out_specs=pl.BlockSpec((M_BLOCK, NODE_DIM), lambda i: (i, 0)),
        out_shape=jax.ShapeDtypeStruct((NUM_NETS, NODE_DIM), jnp.float32),
        compiler_params=pltpu.CompilerParams(
            dimension_semantics=("parallel",),
        ),
    )(mask_i8, emb_bf16)


# speedup vs baseline: 2.9204x; 1.0004x over previous
import jax
import jax.numpy as jnp
from jax.experimental import pallas as pl
from jax.experimental.pallas import tpu as pltpu

NUM_NODES = 10000
NODE_DIM = 128
NUM_NETS = 4096

M_BLOCK = 256


def _mean_net_kernel(mask_ref, emb_ref, out_ref):
    mb = mask_ref[...].astype(jnp.bfloat16)
    sums = jax.lax.dot_general(
        mb, emb_ref[...],
        dimension_numbers=(((1,), (0,)), ((), ())),
        preferred_element_type=jnp.float32,
    )
    counts = jnp.sum(mb, axis=1, keepdims=True, dtype=jnp.float32)
    out_ref[...] = jnp.where(counts > 0, sums / jnp.maximum(counts, 1.0), 0.0)


def kernel(node_embeddings, attention_mask):
    emb_bf16 = node_embeddings.astype(jnp.bfloat16)
    mask_i4 = attention_mask.astype(jnp.int4)
    grid = (NUM_NETS // M_BLOCK,)
    return pl.pallas_call(
        _mean_net_kernel,
        grid=grid,
        in_specs=[
            pl.BlockSpec((M_BLOCK, NUM_NODES), lambda i: (i, 0)),
            pl.BlockSpec((NUM_NODES, NODE_DIM), lambda i: (0, 0)),
        ],
        out_specs=pl.BlockSpec((M_BLOCK, NODE_DIM), lambda i: (i, 0)),
        out_shape=jax.ShapeDtypeStruct((NUM_NETS, NODE_DIM), jnp.float32),
        compiler_params=pltpu.CompilerParams(
            dimension_semantics=("parallel",),
        ),
    )(mask_i4, emb_bf16)
